# baseline (speedup 1.0000x reference)
import functools

import jax
import jax.numpy as jnp
from jax import lax
from jax.experimental import pallas as pl
from jax.experimental.pallas import tpu as pltpu
from jax.experimental.pallas import tpu_sc as plsc

NC = 2
NS = 16
NW = NC * NS
LANES = 16
GSUB = 128


def _sc_unified_gather(idx2d, table, n_feat, batch):
    q, d = table.shape
    chunk = batch // NW
    n_jt = chunk // 128
    n_tr = d // 8
    assert batch % NW == 0 and chunk % GSUB == 0 and chunk % LANES == 0
    assert d % 8 == 0 and batch % 128 == 0 and n_feat % 2 == 0

    mesh = plsc.VectorSubcoreMesh(core_axis_name="c", subcore_axis_name="s")

    @functools.partial(
        pl.kernel,
        mesh=mesh,
        out_type=jax.ShapeDtypeStruct((n_feat, n_tr, batch // 128, 8, 128),
                                      jnp.float32),
        compiler_params=pltpu.CompilerParams(
            use_tc_tiling_on_sc=False, needs_layout_passes=False),
        scratch_types=[
            pltpu.VMEM((chunk,), jnp.int32),
            pltpu.VMEM((chunk,), jnp.int32),
            pltpu.VMEM((chunk, d), jnp.float32),
            pltpu.VMEM((chunk, d), jnp.float32),
            pltpu.VMEM((n_tr, n_jt, 8, 128), jnp.float32),
            pltpu.VMEM((n_tr, n_jt, 8, 128), jnp.float32),
        ] + [pltpu.SemaphoreType.DMA] * 6,
    )
    def sc_kernel(idx_hbm, table_hbm, out_hbm, *rest):
        idxb = rest[0:2]
        rows = rest[2:4]
        tilb = rest[4:6]
        semi = rest[6:8]
        semw = rest[8:10]
        semg = rest[10:12]
        wid = lax.axis_index("s") * NC + lax.axis_index("c")
        base = wid * chunk
        iota = lax.broadcasted_iota(jnp.int32, (LANES,), 0)

        def issue_idx_load(f, b):
            pltpu.async_copy(
                idx_hbm.at[f, pl.ds(base, chunk)], idxb[b], semi[b])

        def wait_idx_load(b):
            pltpu.make_async_copy(
                idx_hbm.at[0, pl.ds(0, chunk)], idxb[b], semi[b]).wait()

        def hash_chunk(f, b):
            salt = f * 7919

            @pl.loop(0, chunk, step=LANES)
            def _(t):
                sl = pl.ds(t, LANES)
                idxb[b][sl] = (idxb[b][sl] * 31 + salt) % q

        def issue_gathers(b):
            @pl.loop(0, chunk, step=GSUB)
            def _(g):
                pltpu.async_copy(
                    table_hbm.at[idxb[b].at[pl.ds(g, GSUB)]],
                    rows[b].at[pl.ds(g, GSUB)],
                    semg[b],
                )

        def wait_gathers(b):
            pltpu.make_async_copy(
                table_hbm.at[pl.ds(0, chunk)], rows[b], semg[b]).wait()

        def transpose_chunk(b):
            @pl.loop(0, d)
            def _(c):
                tr = c >> 3
                s = c & 7
                colv = jnp.full((LANES,), c, jnp.int32)

                @pl.loop(0, n_jt)
                def _(j):
                    rowv = 128 * j + iota

                    @pl.loop(0, 128, step=LANES)
                    def _(lg):
                        v = plsc.load_gather(rows[b], [rowv + lg, colv])
                        tilb[b][tr, j, s, pl.ds(lg, LANES)] = v

        def issue_write(f, b):
            pltpu.async_copy(
                tilb[b],
                out_hbm.at[f, :, pl.ds(wid * n_jt, n_jt)],
                semw[b],
            )

        def wait_write(b):
            pltpu.make_async_copy(
                tilb[b], out_hbm.at[0, :, pl.ds(0, n_jt)], semw[b]).wait()

        issue_idx_load(0, 0)
        issue_idx_load(1, 1)
        wait_idx_load(0)
        hash_chunk(0, 0)
        issue_gathers(0)

        @pl.loop(0, n_feat, step=2)
        def _(i):
            for b in range(2):
                f = i + b

                @pl.when(f < n_feat - 1)
                def _():
                    wait_idx_load(1 - b)
                    hash_chunk(f + 1, 1 - b)
                    issue_gathers(1 - b)

                wait_gathers(b)
                transpose_chunk(b)

                @pl.when(f > 0)
                def _():
                    wait_write(1 - b)

                issue_write(f, b)

                @pl.when(f < n_feat - 2)
                def _():
                    issue_idx_load(f + 2, b)

        wait_write(1)

    return sc_kernel(idx2d, table)


def kernel(inputs, table):
    n_feat, batch, _ = inputs.shape
    d = table.shape[1]
    idx2d = inputs.reshape(n_feat, batch)
    out = _sc_unified_gather(idx2d, table, n_feat, batch)
    return tuple(out[i].transpose(1, 3, 0, 2).reshape(batch, d)
                 for i in range(n_feat))


# speedup vs baseline: 1.0204x; 1.0204x over previous
import functools

import jax
import jax.numpy as jnp
from jax import lax
from jax.experimental import pallas as pl
from jax.experimental.pallas import tpu as pltpu
from jax.experimental.pallas import tpu_sc as plsc

NC = 2
NS = 16
NW = NC * NS
LANES = 16
GSUB = 128


def _sc_unified_gather(idx2d, table, n_feat, batch):
    q, d = table.shape
    chunk = batch // NW
    n_jt = chunk // 128
    n_tr = d // 8
    assert batch % NW == 0 and chunk % GSUB == 0 and chunk % LANES == 0
    assert d % 8 == 0 and batch % 128 == 0 and n_feat % 2 == 0

    mesh = plsc.VectorSubcoreMesh(core_axis_name="c", subcore_axis_name="s")

    @functools.partial(
        pl.kernel,
        mesh=mesh,
        out_type=jax.ShapeDtypeStruct((n_feat, n_tr, batch // 128, 8, 128),
                                      jnp.float32),
        compiler_params=pltpu.CompilerParams(
            use_tc_tiling_on_sc=False, needs_layout_passes=False),
        scratch_types=[
            pltpu.VMEM((chunk,), jnp.int32),
            pltpu.VMEM((chunk,), jnp.int32),
            pltpu.VMEM((chunk, d), jnp.float32),
            pltpu.VMEM((chunk, d), jnp.float32),
            pltpu.VMEM((n_tr, n_jt, 8, 128), jnp.float32),
            pltpu.VMEM((n_tr, n_jt, 8, 128), jnp.float32),
        ] + [pltpu.SemaphoreType.DMA] * 6,
    )
    def sc_kernel(idx_hbm, table_hbm, out_hbm, *rest):
        idxb = rest[0:2]
        rows = rest[2:4]
        tilb = rest[4:6]
        semi = rest[6:8]
        semw = rest[8:10]
        semg = rest[10:12]
        wid = lax.axis_index("s") * NC + lax.axis_index("c")
        base = wid * chunk
        iota = lax.broadcasted_iota(jnp.int32, (LANES,), 0)

        def issue_idx_load(f, b):
            pltpu.async_copy(
                idx_hbm.at[f, pl.ds(base, chunk)], idxb[b], semi[b])

        def wait_idx_load(b):
            pltpu.make_async_copy(
                idx_hbm.at[0, pl.ds(0, chunk)], idxb[b], semi[b]).wait()

        def hash_chunk(f, b):
            salt = f * 7919

            @pl.loop(0, chunk, step=LANES)
            def _(t):
                sl = pl.ds(t, LANES)
                idxb[b][sl] = (idxb[b][sl] * 31 + salt) % q

        def issue_gathers(b):
            @pl.loop(0, chunk, step=GSUB)
            def _(g):
                pltpu.async_copy(
                    table_hbm.at[idxb[b].at[pl.ds(g, GSUB)]],
                    rows[b].at[pl.ds(g, GSUB)],
                    semg[b],
                )

        def wait_gathers(b):
            pltpu.make_async_copy(
                table_hbm.at[pl.ds(0, chunk)], rows[b], semg[b]).wait()

        colvs = [jnp.full((LANES,), c, jnp.int32) for c in range(d)]

        def transpose_chunk(b):
            @pl.loop(0, (chunk // 128) * 8)
            def _(g):
                j = g >> 3
                l0 = (g & 7) * LANES
                rowv = 128 * j + l0 + iota
                for c in range(d):
                    v = plsc.load_gather(rows[b], [rowv, colvs[c]])
                    tilb[b][c >> 3, j, c & 7, pl.ds(l0, LANES)] = v

        def issue_write(f, b):
            pltpu.async_copy(
                tilb[b],
                out_hbm.at[f, :, pl.ds(wid * n_jt, n_jt)],
                semw[b],
            )

        def wait_write(b):
            pltpu.make_async_copy(
                tilb[b], out_hbm.at[0, :, pl.ds(0, n_jt)], semw[b]).wait()

        issue_idx_load(0, 0)
        issue_idx_load(1, 1)
        wait_idx_load(0)
        hash_chunk(0, 0)
        issue_gathers(0)

        @pl.loop(0, n_feat, step=2)
        def _(i):
            for b in range(2):
                f = i + b

                @pl.when(f < n_feat - 1)
                def _():
                    wait_idx_load(1 - b)
                    hash_chunk(f + 1, 1 - b)
                    issue_gathers(1 - b)

                wait_gathers(b)
                transpose_chunk(b)

                @pl.when(f > 0)
                def _():
                    wait_write(1 - b)

                issue_write(f, b)

                @pl.when(f < n_feat - 2)
                def _():
                    issue_idx_load(f + 2, b)

        wait_write(1)

    return sc_kernel(idx2d, table)


def kernel(inputs, table):
    n_feat, batch, _ = inputs.shape
    d = table.shape[1]
    idx2d = inputs.reshape(n_feat, batch)
    out = _sc_unified_gather(idx2d, table, n_feat, batch)
    return tuple(out[i].transpose(1, 3, 0, 2).reshape(batch, d)
                 for i in range(n_feat))
